# baseline (speedup 1.0000x reference)
import functools

import jax
import jax.numpy as jnp
from jax import lax
from jax.experimental import pallas as pl
from jax.experimental.pallas import tpu as pltpu
from jax.experimental.pallas import tpu_sc as plsc

_N = 10000
_E = 320000
_D = 128
_ITERS = 4

_NC = 2
_NS = 16
_NW = _NC * _NS
_EPT = _E // _NW
_B = 50
_NPH = 5
_PB = _EPT // _NPH // _B
_PG = _PB // 8
_DB = 80
_DNB = _EPT // _DB
_RPT0 = 624
_RPT_LAST = _N - (_NS - 1) * _RPT0

_R = 2000
_G = _N // _R

_mesh = plsc.VectorSubcoreMesh(core_axis_name="c", subcore_axis_name="s")


def _agg_body(g_hbm, src_hbm, dst_hbm, out_hbm, acc, src_v, dst_v, rows0,
              rows1, rows2, rows3, gs0, gs1, gs2, gs3, ss0, ss1, ss2, ss3):
    c = lax.axis_index("c")
    s = lax.axis_index("s")
    wid = c * _NS + s

    def _zrow(i, _):
        r = i // (_D // 16)
        k = i % (_D // 16)
        rows0[r, pl.ds(k * 16, 16)] = jnp.zeros((16,), jnp.float32)
        return 0

    pltpu.async_copy(src_hbm.at[wid, 0], src_v, gs0)
    pltpu.async_copy(dst_hbm.at[wid, 0], dst_v, gs1)

    lax.fori_loop(0, _B * (_D // 16), _zrow, 0)
    base = s * _RPT0

    @pl.when(s < _NS - 1)
    def _zero_main():
        for z in range(13):
            pltpu.sync_copy(rows0.at[pl.ds(0, 48)],
                            acc.at[pl.ds(base + z * 48, 48)])

    @pl.when(s == _NS - 1)
    def _zero_last():
        for z in range(16):
            pltpu.sync_copy(rows0.at[pl.ds(0, 40)],
                            acc.at[pl.ds(base + z * 40, 40)])

    plsc.subcore_barrier()

    bufs = (rows0, rows1, rows2, rows3)
    gsems = (gs0, gs1, gs2, gs3)
    ssems = (ss0, ss1, ss2, ss3)
    for ph in range(_NPH):
        if ph > 0:
            pltpu.async_copy(src_hbm.at[wid, ph], src_v, gs0)
            pltpu.async_copy(dst_hbm.at[wid, ph], dst_v, gs1)
        pltpu.make_async_copy(src_hbm.at[wid, ph], src_v, gs0).wait()
        pltpu.make_async_copy(dst_hbm.at[wid, ph], dst_v, gs1).wait()
        for q in range(3):
            pltpu.async_copy(g_hbm.at[src_v.at[q]], bufs[q], gsems[q])

        def _group(grp, _):
            for q in range(8):
                j = grp * 8 + q
                b = q % 4
                pltpu.make_async_copy(g_hbm.at[src_v.at[j]], bufs[b],
                                      gsems[b]).wait()
                pltpu.async_copy(bufs[b], acc.at[dst_v.at[j]], ssems[b],
                                 add=True)
                bn = (b + 3) % 4
                if q == 0:
                    @pl.when(grp >= 1)
                    def _free0():
                        pltpu.make_async_copy(bufs[bn], acc.at[dst_v.at[0]],
                                              ssems[bn]).wait()

                    pltpu.async_copy(g_hbm.at[src_v.at[j + 3]], bufs[bn],
                                     gsems[bn])
                elif q <= 4:
                    pltpu.make_async_copy(bufs[bn], acc.at[dst_v.at[0]],
                                          ssems[bn]).wait()
                    pltpu.async_copy(g_hbm.at[src_v.at[j + 3]], bufs[bn],
                                     gsems[bn])
                else:
                    @pl.when(grp <= _PG - 2)
                    def _recycle():
                        pltpu.make_async_copy(bufs[bn], acc.at[dst_v.at[0]],
                                              ssems[bn]).wait()
                        pltpu.async_copy(g_hbm.at[src_v.at[j + 3]], bufs[bn],
                                         gsems[bn])

            return 0

        lax.fori_loop(0, _PG, _group, 0)

        for q in range(4):
            pltpu.make_async_copy(bufs[q], acc.at[dst_v.at[0]],
                                  ssems[q]).wait()

    plsc.subcore_barrier()

    @pl.when(s < _NS - 1)
    def _dump_main():
        pltpu.sync_copy(acc.at[pl.ds(base, _RPT0)],
                        out_hbm.at[c, pl.ds(base, _RPT0)])

    @pl.when(s == _NS - 1)
    def _dump_last():
        pltpu.sync_copy(acc.at[pl.ds(base, _RPT_LAST)],
                        out_hbm.at[c, pl.ds(base, _RPT_LAST)])


@functools.partial(
    pl.kernel,
    out_type=jax.ShapeDtypeStruct((_NC, _N, _D), jnp.float32),
    mesh=_mesh,
    scratch_types=[
        pltpu.VMEM_SHARED((_N, _D), jnp.float32),
        pltpu.VMEM((_PB, _B), jnp.int32),
        pltpu.VMEM((_PB, _B), jnp.int32),
        pltpu.VMEM((_B, _D), jnp.float32),
        pltpu.VMEM((_B, _D), jnp.float32),
        pltpu.VMEM((_B, _D), jnp.float32),
        pltpu.VMEM((_B, _D), jnp.float32),
        pltpu.SemaphoreType.DMA,
        pltpu.SemaphoreType.DMA,
        pltpu.SemaphoreType.DMA,
        pltpu.SemaphoreType.DMA,
        pltpu.SemaphoreType.DMA,
        pltpu.SemaphoreType.DMA,
        pltpu.SemaphoreType.DMA,
        pltpu.SemaphoreType.DMA,
    ],
)
def _agg(g_hbm, src_hbm, dst_hbm, out_hbm, acc, src_v, dst_v, rows0, rows1,
         rows2, rows3, gs0, gs1, gs2, gs3, ss0, ss1, ss2, ss3):
    _agg_body(g_hbm, src_hbm, dst_hbm, out_hbm, acc, src_v, dst_v, rows0,
              rows1, rows2, rows3, gs0, gs1, gs2, gs3, ss0, ss1, ss2, ss3)


def _deg_body(dst_hbm, out_hbm, acc, dst_v, ones_v, zero_v):
    c = lax.axis_index("c")
    s = lax.axis_index("s")
    wid = c * _NS + s

    for k in range(_DB // 16):
        ones_v[pl.ds(k * 16, 16)] = jnp.ones((16,), jnp.float32)

    def _z(i, _):
        zero_v[pl.ds(i * 16, 16)] = jnp.zeros((16,), jnp.float32)
        return 0

    lax.fori_loop(0, 640 // 16, _z, 0)

    @pl.when(s < _NS - 1)
    def _z15():
        pltpu.sync_copy(zero_v.at[pl.ds(0, _RPT0)],
                        acc.at[pl.ds(s * _RPT0, _RPT0)])

    @pl.when(s == _NS - 1)
    def _zlast():
        pltpu.sync_copy(zero_v, acc.at[pl.ds((_NS - 1) * _RPT0, _RPT_LAST)])

    pltpu.sync_copy(dst_hbm.at[wid], dst_v)
    plsc.subcore_barrier()

    def _batch(j, _):
        pltpu.sync_copy(ones_v, acc.at[dst_v.at[j]], add=True)
        return 0

    lax.fori_loop(0, _DNB, _batch, 0)

    plsc.subcore_barrier()

    @pl.when(s < _NS - 1)
    def _d15():
        pltpu.sync_copy(acc.at[pl.ds(s * _RPT0, _RPT0)],
                        zero_v.at[pl.ds(0, _RPT0)])
        pltpu.sync_copy(zero_v.at[pl.ds(0, _RPT0)],
                        out_hbm.at[pl.ds(c * _N + s * _RPT0, _RPT0)])

    @pl.when(s == _NS - 1)
    def _dlast():
        pltpu.sync_copy(acc.at[pl.ds((_NS - 1) * _RPT0, _RPT_LAST)], zero_v)
        pltpu.sync_copy(zero_v,
                        out_hbm.at[pl.ds(c * _N + (_NS - 1) * _RPT0,
                                         _RPT_LAST)])


@functools.partial(
    pl.kernel,
    out_type=jax.ShapeDtypeStruct((_NC * _N,), jnp.float32),
    mesh=_mesh,
    scratch_types=[
        pltpu.VMEM_SHARED((_N,), jnp.float32),
        pltpu.VMEM((_DNB, _DB), jnp.int32),
        pltpu.VMEM((_DB,), jnp.float32),
        pltpu.VMEM((640,), jnp.float32),
    ],
)
def _deg(dst_hbm, out_hbm, acc, dst_v, ones_v, zero_v):
    _deg_body(dst_hbm, out_hbm, acc, dst_v, ones_v, zero_v)


def _ench_body(x_ref, we_ref, be_ref, h_ref):
    h_ref[...] = jnp.maximum(
        jnp.dot(x_ref[...], we_ref[...], preferred_element_type=jnp.float32)
        + be_ref[...], 0.0)


def _encg_body(h_ref, d0_ref, d1_ref, wg_ref, g_ref, dinv_ref):
    deg = d0_ref[...] + d1_ref[...] + 1.0
    dinv = lax.rsqrt(deg)
    dinv_ref[...] = dinv
    g_ref[...] = jnp.dot(h_ref[...] * dinv, wg_ref[...],
                         preferred_element_type=jnp.float32)


def _blend(h, p0, p1, g, dinv, bg, sf):
    new_h = jnp.maximum(dinv * (p0 + p1 + g) + bg, 0.0)
    return sf * h + (1.0 - sf) * new_h


def _mid_body(i, h_ref, g_ref, p0_ref, p1_ref, dinv_ref, bg_ref, sch_ref,
              wg_ref, hn_ref, gn_ref):
    sf = 1.0 / (1.0 + jnp.exp(-sch_ref[0, i]))
    dinv = dinv_ref[...]
    hn = _blend(h_ref[...], p0_ref[...], p1_ref[...], g_ref[...], dinv,
                bg_ref[...], sf)
    hn_ref[...] = hn
    gn_ref[...] = jnp.dot(hn * dinv, wg_ref[...],
                          preferred_element_type=jnp.float32)


def _dec_body(i, h_ref, g_ref, p0_ref, p1_ref, dinv_ref, bg_ref, sch_ref,
              wd_ref, bd_ref, y_ref):
    sf = 1.0 / (1.0 + jnp.exp(-sch_ref[0, i]))
    hn = _blend(h_ref[...], p0_ref[...], p1_ref[...], g_ref[...],
                dinv_ref[...], bg_ref[...], sf)
    y_ref[...] = jnp.dot(hn, wd_ref[...],
                         preferred_element_type=jnp.float32) + bd_ref[...]


_row_spec = pl.BlockSpec((_R, _D), lambda i: (i, 0))
_mat_spec = pl.BlockSpec((_D, _D), lambda i: (0, 0))
_vec_spec = pl.BlockSpec((1, _D), lambda i: (0, 0))
_col_spec = pl.BlockSpec((_R, 1), lambda i: (i, 0))
_smem_spec = pl.BlockSpec(memory_space=pltpu.MemorySpace.SMEM)


def _ench_call(x, We, be):
    return pl.pallas_call(
        _ench_body,
        grid=(_G,),
        in_specs=[_row_spec, _mat_spec, _vec_spec],
        out_specs=_row_spec,
        out_shape=jax.ShapeDtypeStruct((_N, _D), jnp.float32),
    )(x, We, be)


def _encg_call(h, d0, d1, Wg):
    return pl.pallas_call(
        _encg_body,
        grid=(_G,),
        in_specs=[_row_spec, _col_spec, _col_spec, _mat_spec],
        out_specs=[_row_spec, _col_spec],
        out_shape=[jax.ShapeDtypeStruct((_N, _D), jnp.float32),
                   jax.ShapeDtypeStruct((_N, 1), jnp.float32)],
    )(h, d0, d1, Wg)


def _mid_call(i, h, g, p0, p1, dinv, bg, sch, Wg):
    return pl.pallas_call(
        functools.partial(_mid_body, i),
        grid=(_G,),
        in_specs=[_row_spec, _row_spec, _row_spec, _row_spec, _col_spec,
                  _vec_spec, _smem_spec, _mat_spec],
        out_specs=[_row_spec, _row_spec],
        out_shape=[jax.ShapeDtypeStruct((_N, _D), jnp.float32),
                   jax.ShapeDtypeStruct((_N, _D), jnp.float32)],
    )(h, g, p0, p1, dinv, bg, sch, Wg)


def _dec_call(i, h, g, p0, p1, dinv, bg, sch, Wd, bd):
    return pl.pallas_call(
        functools.partial(_dec_body, i),
        grid=(_G,),
        in_specs=[_row_spec, _row_spec, _row_spec, _row_spec, _col_spec,
                  _vec_spec, _smem_spec, _mat_spec, _vec_spec],
        out_specs=_row_spec,
        out_shape=jax.ShapeDtypeStruct((_N, _D), jnp.float32),
    )(h, g, p0, p1, dinv, bg, sch, Wd, bd)


def kernel(x, edge_index, W_enc, b_enc, W_gc, b_gc, schedule, W_dec, b_dec):
    src4 = edge_index[0].reshape(_NW, _NPH, _PB, _B)
    dst4 = edge_index[1].reshape(_NW, _NPH, _PB, _B)
    dst3 = edge_index[1].reshape(_NW, _DNB, _DB)

    degp = _deg(dst3).reshape(_NC, _N)
    d0 = degp[0].reshape(_N, 1)
    d1 = degp[1].reshape(_N, 1)

    be = b_enc.reshape(1, _D)
    bg = b_gc.reshape(1, _D)
    bd = b_dec.reshape(1, _D)
    sch = schedule.reshape(1, _ITERS)

    h = _ench_call(x, W_enc, be)
    g, dinv = _encg_call(h, d0, d1, W_gc)

    for i in range(_ITERS):
        p = _agg(g, src4, dst4)
        if i < _ITERS - 1:
            h, g = _mid_call(i, h, g, p[0], p[1], dinv, bg, sch, W_gc)
        else:
            return _dec_call(i, h, g, p[0], p[1], dinv, bg, sch, W_dec, bd)


# speedup vs baseline: 1.0076x; 1.0076x over previous
import functools

import jax
import jax.numpy as jnp
from jax import lax
from jax.experimental import pallas as pl
from jax.experimental.pallas import tpu as pltpu
from jax.experimental.pallas import tpu_sc as plsc

_N = 10000
_E = 320000
_D = 128
_ITERS = 4

_NC = 2
_NS = 16
_NW = _NC * _NS
_EPT = _E // _NW
_B = 50
_NPH = 5
_PB = _EPT // _NPH // _B
_PG = _PB // 8
_DB = 80
_DNB = _EPT // _DB
_RPT0 = 624
_RPT_LAST = _N - (_NS - 1) * _RPT0

_R = 2000
_G = _N // _R

_mesh = plsc.VectorSubcoreMesh(core_axis_name="c", subcore_axis_name="s")


def _agg_body(g_hbm, src_hbm, dst_hbm, out_hbm, acc, src_v, dst_v, rows0,
              rows1, rows2, rows3, gs0, gs1, gs2, gs3, ss0, ss1, ss2, ss3):
    c = lax.axis_index("c")
    s = lax.axis_index("s")
    wid = c * _NS + s

    def _zrow(i, _):
        r = i // (_D // 16)
        k = i % (_D // 16)
        rows0[r, pl.ds(k * 16, 16)] = jnp.zeros((16,), jnp.float32)
        return 0

    pltpu.async_copy(src_hbm.at[wid, 0], src_v, gs0)
    pltpu.async_copy(dst_hbm.at[wid, 0], dst_v, gs1)

    lax.fori_loop(0, _B * (_D // 16), _zrow, 0)
    base = s * _RPT0

    @pl.when(s < _NS - 1)
    def _zero_main():
        for z in range(13):
            pltpu.sync_copy(rows0.at[pl.ds(0, 48)],
                            acc.at[pl.ds(base + z * 48, 48)])

    @pl.when(s == _NS - 1)
    def _zero_last():
        for z in range(16):
            pltpu.sync_copy(rows0.at[pl.ds(0, 40)],
                            acc.at[pl.ds(base + z * 40, 40)])

    plsc.subcore_barrier()

    bufs = (rows0, rows1, rows2, rows3)
    gsems = (gs0, gs1, gs2, gs3)
    ssems = (ss0, ss1, ss2, ss3)
    for ph in range(_NPH):
        if ph > 0:
            pltpu.async_copy(src_hbm.at[wid, ph], src_v, gs0)
            pltpu.async_copy(dst_hbm.at[wid, ph], dst_v, gs1)
        pltpu.make_async_copy(src_hbm.at[wid, ph], src_v, gs0).wait()
        pltpu.make_async_copy(dst_hbm.at[wid, ph], dst_v, gs1).wait()
        for q in range(3):
            pltpu.async_copy(g_hbm.at[src_v.at[q]], bufs[q], gsems[q])

        def _group(grp, _):
            for q in range(8):
                j = grp * 8 + q
                b = q % 4
                pltpu.make_async_copy(g_hbm.at[src_v.at[j]], bufs[b],
                                      gsems[b]).wait()
                pltpu.async_copy(bufs[b], acc.at[dst_v.at[j]], ssems[b],
                                 add=True)
                bn = (b + 3) % 4
                if q == 0:
                    @pl.when(grp >= 1)
                    def _free0():
                        pltpu.make_async_copy(bufs[bn], acc.at[dst_v.at[0]],
                                              ssems[bn]).wait()

                    pltpu.async_copy(g_hbm.at[src_v.at[j + 3]], bufs[bn],
                                     gsems[bn])
                elif q <= 4:
                    pltpu.make_async_copy(bufs[bn], acc.at[dst_v.at[0]],
                                          ssems[bn]).wait()
                    pltpu.async_copy(g_hbm.at[src_v.at[j + 3]], bufs[bn],
                                     gsems[bn])
                else:
                    @pl.when(grp <= _PG - 2)
                    def _recycle():
                        pltpu.make_async_copy(bufs[bn], acc.at[dst_v.at[0]],
                                              ssems[bn]).wait()
                        pltpu.async_copy(g_hbm.at[src_v.at[j + 3]], bufs[bn],
                                         gsems[bn])

            return 0

        lax.fori_loop(0, _PG, _group, 0)

        for q in range(4):
            pltpu.make_async_copy(bufs[q], acc.at[dst_v.at[0]],
                                  ssems[q]).wait()

    plsc.subcore_barrier()

    @pl.when(s < _NS - 1)
    def _dump_main():
        pltpu.sync_copy(acc.at[pl.ds(base, _RPT0)],
                        out_hbm.at[c, pl.ds(base, _RPT0)])

    @pl.when(s == _NS - 1)
    def _dump_last():
        pltpu.sync_copy(acc.at[pl.ds(base, _RPT_LAST)],
                        out_hbm.at[c, pl.ds(base, _RPT_LAST)])


@functools.partial(
    pl.kernel,
    out_type=jax.ShapeDtypeStruct((_NC, _N, _D), jnp.float32),
    mesh=_mesh,
    scratch_types=[
        pltpu.VMEM_SHARED((_N, _D), jnp.float32),
        pltpu.VMEM((_PB, _B), jnp.int32),
        pltpu.VMEM((_PB, _B), jnp.int32),
        pltpu.VMEM((_B, _D), jnp.float32),
        pltpu.VMEM((_B, _D), jnp.float32),
        pltpu.VMEM((_B, _D), jnp.float32),
        pltpu.VMEM((_B, _D), jnp.float32),
        pltpu.SemaphoreType.DMA,
        pltpu.SemaphoreType.DMA,
        pltpu.SemaphoreType.DMA,
        pltpu.SemaphoreType.DMA,
        pltpu.SemaphoreType.DMA,
        pltpu.SemaphoreType.DMA,
        pltpu.SemaphoreType.DMA,
        pltpu.SemaphoreType.DMA,
    ],
)
def _agg(g_hbm, src_hbm, dst_hbm, out_hbm, acc, src_v, dst_v, rows0, rows1,
         rows2, rows3, gs0, gs1, gs2, gs3, ss0, ss1, ss2, ss3):
    _agg_body(g_hbm, src_hbm, dst_hbm, out_hbm, acc, src_v, dst_v, rows0,
              rows1, rows2, rows3, gs0, gs1, gs2, gs3, ss0, ss1, ss2, ss3)


def _deg_body(dst_hbm, out_hbm, acc, dst_v, ones_v, zero_v):
    c = lax.axis_index("c")
    s = lax.axis_index("s")
    wid = c * _NS + s

    for k in range(_DB // 16):
        ones_v[pl.ds(k * 16, 16)] = jnp.ones((16,), jnp.float32)

    def _z(i, _):
        zero_v[pl.ds(i * 16, 16)] = jnp.zeros((16,), jnp.float32)
        return 0

    lax.fori_loop(0, 640 // 16, _z, 0)

    @pl.when(s < _NS - 1)
    def _z15():
        pltpu.sync_copy(zero_v.at[pl.ds(0, _RPT0)],
                        acc.at[pl.ds(s * _RPT0, _RPT0)])

    @pl.when(s == _NS - 1)
    def _zlast():
        pltpu.sync_copy(zero_v, acc.at[pl.ds((_NS - 1) * _RPT0, _RPT_LAST)])

    pltpu.sync_copy(dst_hbm.at[wid], dst_v)
    plsc.subcore_barrier()

    def _batch(j, _):
        pltpu.sync_copy(ones_v, acc.at[dst_v.at[j]], add=True)
        return 0

    lax.fori_loop(0, _DNB, _batch, 0)

    plsc.subcore_barrier()

    @pl.when(s < _NS - 1)
    def _d15():
        pltpu.sync_copy(acc.at[pl.ds(s * _RPT0, _RPT0)],
                        zero_v.at[pl.ds(0, _RPT0)])
        pltpu.sync_copy(zero_v.at[pl.ds(0, _RPT0)],
                        out_hbm.at[pl.ds(c * _N + s * _RPT0, _RPT0)])

    @pl.when(s == _NS - 1)
    def _dlast():
        pltpu.sync_copy(acc.at[pl.ds((_NS - 1) * _RPT0, _RPT_LAST)], zero_v)
        pltpu.sync_copy(zero_v,
                        out_hbm.at[pl.ds(c * _N + (_NS - 1) * _RPT0,
                                         _RPT_LAST)])


@functools.partial(
    pl.kernel,
    out_type=jax.ShapeDtypeStruct((_NC * _N,), jnp.float32),
    mesh=_mesh,
    scratch_types=[
        pltpu.VMEM_SHARED((_N,), jnp.float32),
        pltpu.VMEM((_DNB, _DB), jnp.int32),
        pltpu.VMEM((_DB,), jnp.float32),
        pltpu.VMEM((640,), jnp.float32),
    ],
)
def _deg(dst_hbm, out_hbm, acc, dst_v, ones_v, zero_v):
    _deg_body(dst_hbm, out_hbm, acc, dst_v, ones_v, zero_v)


def _enc_body(x_ref, we_ref, be_ref, d0_ref, d1_ref, wg_ref,
              h_ref, g_ref, dinv_ref):
    h = jnp.maximum(
        jnp.dot(x_ref[...], we_ref[...], preferred_element_type=jnp.float32)
        + be_ref[...], 0.0)
    deg = d0_ref[...] + d1_ref[...] + 1.0
    dinv = lax.rsqrt(deg)
    h_ref[...] = h
    dinv_ref[...] = dinv
    g_ref[...] = jnp.dot(h * dinv, wg_ref[...],
                         preferred_element_type=jnp.float32)


def _blend(h, p0, p1, g, dinv, bg, sf):
    new_h = jnp.maximum(dinv * (p0 + p1 + g) + bg, 0.0)
    return sf * h + (1.0 - sf) * new_h


def _mid_body(i, h_ref, g_ref, p0_ref, p1_ref, dinv_ref, bg_ref, sch_ref,
              wg_ref, hn_ref, gn_ref):
    sf = 1.0 / (1.0 + jnp.exp(-sch_ref[0, i]))
    dinv = dinv_ref[...]
    hn = _blend(h_ref[...], p0_ref[...], p1_ref[...], g_ref[...], dinv,
                bg_ref[...], sf)
    hn_ref[...] = hn
    gn_ref[...] = jnp.dot(hn * dinv, wg_ref[...],
                          preferred_element_type=jnp.float32)


def _dec_body(i, h_ref, g_ref, p0_ref, p1_ref, dinv_ref, bg_ref, sch_ref,
              wd_ref, bd_ref, y_ref):
    sf = 1.0 / (1.0 + jnp.exp(-sch_ref[0, i]))
    hn = _blend(h_ref[...], p0_ref[...], p1_ref[...], g_ref[...],
                dinv_ref[...], bg_ref[...], sf)
    y_ref[...] = jnp.dot(hn, wd_ref[...],
                         preferred_element_type=jnp.float32) + bd_ref[...]


_row_spec = pl.BlockSpec((_R, _D), lambda i: (i, 0))
_mat_spec = pl.BlockSpec((_D, _D), lambda i: (0, 0))
_vec_spec = pl.BlockSpec((1, _D), lambda i: (0, 0))
_col_spec = pl.BlockSpec((_R, 1), lambda i: (i, 0))
_smem_spec = pl.BlockSpec(memory_space=pltpu.MemorySpace.SMEM)


def _enc_call(x, We, be, d0, d1, Wg):
    return pl.pallas_call(
        _enc_body,
        grid=(_G,),
        in_specs=[_row_spec, _mat_spec, _vec_spec, _col_spec, _col_spec,
                  _mat_spec],
        out_specs=[_row_spec, _row_spec, _col_spec],
        out_shape=[jax.ShapeDtypeStruct((_N, _D), jnp.float32),
                   jax.ShapeDtypeStruct((_N, _D), jnp.float32),
                   jax.ShapeDtypeStruct((_N, 1), jnp.float32)],
    )(x, We, be, d0, d1, Wg)


def _mid_call(i, h, g, p0, p1, dinv, bg, sch, Wg):
    return pl.pallas_call(
        functools.partial(_mid_body, i),
        grid=(_G,),
        in_specs=[_row_spec, _row_spec, _row_spec, _row_spec, _col_spec,
                  _vec_spec, _smem_spec, _mat_spec],
        out_specs=[_row_spec, _row_spec],
        out_shape=[jax.ShapeDtypeStruct((_N, _D), jnp.float32),
                   jax.ShapeDtypeStruct((_N, _D), jnp.float32)],
    )(h, g, p0, p1, dinv, bg, sch, Wg)


def _dec_call(i, h, g, p0, p1, dinv, bg, sch, Wd, bd):
    return pl.pallas_call(
        functools.partial(_dec_body, i),
        grid=(_G,),
        in_specs=[_row_spec, _row_spec, _row_spec, _row_spec, _col_spec,
                  _vec_spec, _smem_spec, _mat_spec, _vec_spec],
        out_specs=_row_spec,
        out_shape=jax.ShapeDtypeStruct((_N, _D), jnp.float32),
    )(h, g, p0, p1, dinv, bg, sch, Wd, bd)


def kernel(x, edge_index, W_enc, b_enc, W_gc, b_gc, schedule, W_dec, b_dec):
    src4 = edge_index[0].reshape(_NW, _NPH, _PB, _B)
    dst4 = edge_index[1].reshape(_NW, _NPH, _PB, _B)
    dst3 = edge_index[1].reshape(_NW, _DNB, _DB)

    degp = _deg(dst3).reshape(_NC, _N)
    d0 = degp[0].reshape(_N, 1)
    d1 = degp[1].reshape(_N, 1)

    be = b_enc.reshape(1, _D)
    bg = b_gc.reshape(1, _D)
    bd = b_dec.reshape(1, _D)
    sch = schedule.reshape(1, _ITERS)

    h, g, dinv = _enc_call(x, W_enc, be, d0, d1, W_gc)

    for i in range(_ITERS):
        p = _agg(g, src4, dst4)
        if i < _ITERS - 1:
            h, g = _mid_call(i, h, g, p[0], p[1], dinv, bg, sch, W_gc)
        else:
            return _dec_call(i, h, g, p[0], p[1], dinv, bg, sch, W_dec, bd)
